# baseline (speedup 1.0000x reference)
import functools

import jax
import jax.numpy as jnp
from jax import lax
from jax.experimental import pallas as pl
from jax.experimental.pallas import tpu as pltpu
from jax.experimental.pallas import tpu_sc as plsc

VOCAB = 28996
DIM = 768


_BLK = 1024


def _stats_body(emb_ref, sum_ref, sq_ref):
    i = pl.program_id(0)
    x = emb_ref[...]
    row = lax.broadcasted_iota(jnp.int32, x.shape, 0) + i * _BLK
    x = jnp.where(row < VOCAB, x, 0.0)
    s = jnp.sum(x, axis=0, keepdims=True)
    q = jnp.sum(x * x, axis=0, keepdims=True)

    @pl.when(i == 0)
    def _():
        sum_ref[...] = s
        sq_ref[...] = q

    @pl.when(i > 0)
    def _():
        sum_ref[...] += s
        sq_ref[...] += q


def _column_stats(embeddings):
    grid = (VOCAB + _BLK - 1) // _BLK
    s, q = pl.pallas_call(
        _stats_body,
        grid=(grid,),
        in_specs=[pl.BlockSpec((_BLK, DIM), lambda i: (i, 0))],
        out_specs=[
            pl.BlockSpec((1, DIM), lambda i: (0, 0)),
            pl.BlockSpec((1, DIM), lambda i: (0, 0)),
        ],
        out_shape=[
            jax.ShapeDtypeStruct((1, DIM), jnp.float32),
            jax.ShapeDtypeStruct((1, DIM), jnp.float32),
        ],
    )(embeddings)
    n = jnp.float32(VOCAB)
    mean = s[0] / n
    var = (q[0] - s[0] * s[0] / n) / (n - 1.0)
    rstd = lax.rsqrt(var)
    return mean, rstd


_NW = 32
_L = 16
_RB = 2


def _make_gather_norm(BATCH, T):
    bpw = BATCH // _NW
    nch = bpw // _RB
    ipc = _RB * T
    mesh = plsc.VectorSubcoreMesh(core_axis_name="c", subcore_axis_name="s")

    @functools.partial(
        pl.kernel,
        mesh=mesh,
        out_type=jax.ShapeDtypeStruct((BATCH, T, DIM), jnp.float32),
        scratch_types=[
            pltpu.VMEM((bpw * T,), jnp.int32),
            pltpu.VMEM((DIM,), jnp.float32),
            pltpu.VMEM((DIM,), jnp.float32),
            pltpu.VMEM((ipc, DIM), jnp.float32),
            pltpu.VMEM((_RB, T, DIM), jnp.float32),
            pltpu.SemaphoreType.DMA,
        ],
    )
    def gather_norm(table_hbm, ids_hbm, mean_hbm, rstd_hbm, out_hbm,
                    idx_v, mean_v, rstd_v, gbuf, obuf, gsem):
        wid = lax.axis_index("s") * 2 + lax.axis_index("c")
        base = wid * bpw
        pltpu.sync_copy(ids_hbm.at[pl.ds(base * T, bpw * T)], idx_v)
        pltpu.sync_copy(mean_hbm, mean_v)
        pltpu.sync_copy(rstd_hbm, rstd_v)

        def normalize(gb, ob):
            for r in range(_RB):
                def col(j, carry):
                    mj = mean_v[pl.ds(j * _L, _L)]
                    rj = rstd_v[pl.ds(j * _L, _L)]

                    def row(t, carry2):
                        x = gb[r * T + t, pl.ds(j * _L, _L)]
                        ob[r, t, pl.ds(j * _L, _L)] = (x - mj) * rj
                        return carry2

                    return lax.fori_loop(0, T, row, carry, unroll=4)

                lax.fori_loop(0, DIM // _L, col, 0, unroll=2)

        def step(c, carry):
            pltpu.async_copy(table_hbm.at[idx_v.at[pl.ds(c * ipc, ipc)]],
                             gbuf, gsem).wait()
            normalize(gbuf, obuf)
            pltpu.sync_copy(obuf, out_hbm.at[pl.ds(base + c * _RB, _RB)])
            return carry

        lax.fori_loop(0, nch, step, 0)

    return gather_norm


def kernel(input_ids, embeddings):
    ids = input_ids.reshape(-1).astype(jnp.int32)
    BATCH, T = input_ids.shape
    mean, rstd = _column_stats(embeddings)
    out = _make_gather_norm(BATCH, T)(embeddings, ids, mean, rstd)
    return out
